# baseline (speedup 1.0000x reference)
import functools

import jax
import jax.numpy as jnp
from jax import lax
from jax.experimental import pallas as pl
from jax.experimental.pallas import tpu as pltpu
from jax.experimental.pallas import tpu_sc as plsc

N = 10000
E = 160000
T = 5
DS = 4
H = 64
K = 4
DH = 64
TEMP = 0.5

NC = 2
NS = 16
NW = NC * NS
EP = 163840
NACC = 10240
BE = 2048
BN = 2048

_f32 = jnp.float32
_bf16 = jnp.bfloat16


def _relu(v):
    return jnp.maximum(v, 0.0)


def _dot(a, b):
    return jnp.dot(a.astype(_bf16), b.astype(_bf16),
                   preferred_element_type=_f32)


def _dotf(a, b):
    return jnp.dot(a, b, preferred_element_type=_f32)


def _full_spec(shape):
    return pl.BlockSpec(shape, lambda i: (0,) * len(shape))


def _sc_gather(tab, idx, d, nch, gch, dtype=_f32):
    rows_w = nch * 128
    grows = gch * 128
    ngroups = nch // gch
    mesh = plsc.VectorSubcoreMesh(core_axis_name="c", subcore_axis_name="s")

    @functools.partial(
        pl.kernel,
        out_type=jax.ShapeDtypeStruct((NW * rows_w, d), dtype),
        mesh=mesh,
        compiler_params=pltpu.CompilerParams(use_tc_tiling_on_sc=False),
        scratch_types=[
            pltpu.VMEM((nch, 128), jnp.int32),
            pltpu.VMEM((grows, d), dtype),
            pltpu.SemaphoreType.DMA,
        ],
    )
    def body(tab_h, idx_h, out_h, idxv, rowsv, sem):
        wid = lax.axis_index("s") * NC + lax.axis_index("c")
        pltpu.sync_copy(idx_h.at[wid], idxv)
        base = wid * rows_w

        def per_group(g, carry):
            descs = []
            for c in range(gch):
                descs.append(pltpu.async_copy(
                    tab_h.at[idxv.at[g * gch + c]],
                    rowsv.at[pl.ds(c * 128, 128)], sem))
            for dsc in descs:
                dsc.wait()
            pltpu.sync_copy(rowsv, out_h.at[pl.ds(base + g * grows, grows)])
            return carry

        lax.fori_loop(0, ngroups, per_group, 0)

    return body(tab, idx)


def _sc_scatter(msgs, idx, zeros_acc):
    tn = msgs.shape[0]
    rows_w = EP // NW
    acc_sl = NACC // NS
    mesh = plsc.VectorSubcoreMesh(core_axis_name="c", subcore_axis_name="s")

    @functools.partial(
        pl.kernel,
        out_type=jax.ShapeDtypeStruct((NC, tn, NACC, 128), _f32),
        mesh=mesh,
        compiler_params=pltpu.CompilerParams(use_tc_tiling_on_sc=False),
        scratch_types=[
            pltpu.VMEM((40, 128), jnp.int32),
            pltpu.VMEM((256, 128), _f32),
            pltpu.VMEM_SHARED((NACC, 128), _f32),
        ],
    )
    def body(m_h, idx_h, z_h, out_h, idxv, datav, acc):
        cid = lax.axis_index("c")
        sid = lax.axis_index("s")
        wid = sid * NC + cid
        pltpu.sync_copy(idx_h.at[wid], idxv)
        base = wid * rows_w

        def per_t(t, carry):
            pltpu.sync_copy(z_h.at[pl.ds(sid * acc_sl, acc_sl)],
                            acc.at[pl.ds(sid * acc_sl, acc_sl)])
            plsc.subcore_barrier()

            def per_group(g, c2):
                pltpu.sync_copy(m_h.at[t].at[pl.ds(base + g * 256, 256)],
                                datav)
                for c in range(2):
                    pltpu.sync_copy(datav.at[pl.ds(c * 128, 128)],
                                    acc.at[idxv.at[g * 2 + c]], add=True)
                return c2

            lax.fori_loop(0, 20, per_group, 0)
            plsc.subcore_barrier()
            pltpu.sync_copy(acc.at[pl.ds(sid * acc_sl, acc_sl)],
                            out_h.at[cid].at[t].at[pl.ds(sid * acc_sl, acc_sl)])
            plsc.subcore_barrier()
            return carry

        lax.fori_loop(0, tn, per_t, 0)

    return body(msgs, idx, zeros_acc)


def _enc1_body(xgd, xgs, ea,
               wnp, bnp, wep, bep,
               a1, b1, c1, be1, we2, be2,
               d1, d2, bn1, wn2, bn2,
               ea2_o, m_o):
    xd = xgd[...]
    xs_ = xgs[...]
    eaa = ea[...]
    for t in range(T):
        sl = slice(4 * t, 4 * t + 4)
        xi = _relu(_dot(xd[:, sl], wnp[...]) + bnp[...])
        xj = _relu(_dot(xs_[:, sl], wnp[...]) + bnp[...])
        he = _relu(_dot(eaa[:, sl], wep[...]) + bep[...])
        h = _relu(_dot(xi, a1[...]) + _dot(xj, b1[...]) + _dot(he, c1[...])
                  + be1[...])
        ea2 = _dot(h, we2[...]) + be2[...]
        mm = _dot(_relu(_dot(xi, d1[...]) + _dot(ea2, d2[...]) + bn1[...]),
                  wn2[...]) + bn2[...]
        ea2_o[t] = ea2.astype(_bf16)
        m_o[t] = mm


def _mega_body(ea2, g0d, g0s, g1d, g1s, g2d, g2s, g3d, g3s, g4d, g4s,
               xgd, xgs, gum,
               a2, b2, c2, be1, we2, be2, wh, bh,
               wirz_f, win_f, whrz_f, whn_f, brz_f, bin_f, bhn_f,
               wirz_b, win_b, whrz_b, whn_b, brz_b, bin_b, bhn_b,
               wof, wob, bo,
               w1d, w1s, b1c, w2bd, b2c, rep, summ,
               msg_o):
    gds = [g0d, g1d, g2d, g3d, g4d]
    gss = [g0s, g1s, g2s, g3s, g4s]
    logits = []
    for t in range(T):
        xi2 = gds[t][...]
        xj2 = gss[t][...]
        e2t = ea2[t]
        h = _relu(_dot(xi2, a2[...]) + _dot(xj2, b2[...]) + _dot(e2t, c2[...])
                  + be1[...])
        ea3 = _dot(h, we2[...]) + be2[...]
        logits.append(_dot(ea3, wh[...]) + bh[...])

    def gru_step(xt, hprev, wirz, win, whrz, whn, brz, binn, bhn):
        srz = jax.nn.sigmoid(_dot(xt, wirz[...]) + _dot(hprev, whrz[...])
                             + brz[...])
        r = srz[:, :H]
        z = srz[:, H:]
        c = jnp.tanh(_dot(xt, win[...]) + binn[...]
                     + r * (_dot(hprev, whn[...]) + bhn[...]))
        return (1.0 - z) * c + z * hprev

    nb = logits[0].shape[0]
    hf = jnp.zeros((nb, H), _f32)
    yf = []
    for t in range(T):
        hf = gru_step(logits[t], hf, wirz_f, win_f, whrz_f, whn_f,
                      brz_f, bin_f, bhn_f)
        yf.append(hf)
    hb = jnp.zeros((nb, H), _f32)
    yb = [None] * T
    for t in reversed(range(T)):
        hb = gru_step(logits[t], hb, wirz_b, win_b, whrz_b, whn_b,
                      brz_b, bin_b, bhn_b)
        yb[t] = hb

    xd = xgd[...]
    xs_ = xgs[...]
    gu = gum[...]
    for t in range(T - 1):
        sm = _dot(yf[t], wof[...]) + _dot(yb[t], wob[...]) + bo[...]
        a = (sm + gu[:, 4 * t:4 * t + 4]) * (1.0 / TEMP)
        ex = jnp.exp(a)
        pr = ex / jnp.sum(ex, axis=1, keepdims=True)
        pbig = _dotf(pr, rep[...])
        sl = slice(4 * t, 4 * t + 4)
        xdt = xd[:, sl]
        xst = xs_[:, sl]
        h1 = _relu(_dot(xdt, w1d[...]) + _dot(xst, w1s[...]) + b1c[...])
        hc = _relu(_dot(h1, w2bd[...]) + b2c[...])
        msg_o[t] = _dotf(hc * pbig, summ[...])


def _addp_body(p, o):
    o[...] = (p[0, 0, :, :64] + p[1, 0, :, :64]).astype(_bf16)


def _node_body(xr, p2, o1a, o1b, bo1, w2, bo2, w3, bo3, out):
    xx = xr[...]
    for t in range(T - 1):
        aggr = p2[0, t, :, :64] + p2[1, t, :, :64]
        xt = xx[:, 4 * t:4 * t + 4]
        h = _relu(_dotf(xt, o1a[...]) + _dotf(aggr, o1b[...]) + bo1[...])
        h = _relu(_dotf(h, w2[...]) + bo2[...])
        delta = _dotf(h, w3[...]) + bo3[...]
        out[:, t, :] = xt + delta


def _edge_spec_d(be):
    return pl.BlockSpec((be, 32), lambda i: (i, 0))


def _edge_spec_s(be):
    return pl.BlockSpec((be, 32), lambda i: (EP // be + i, 0))


def _enc1(xg, eaP, ws):
    grid = (EP // BE,)
    in_specs = [_edge_spec_d(BE), _edge_spec_s(BE),
                pl.BlockSpec((BE, 20), lambda i: (i, 0))]
    in_specs += [_full_spec(w.shape) for w in ws]
    out_specs = [pl.BlockSpec((T, BE, 64), lambda i: (0, i, 0)),
                 pl.BlockSpec((T, BE, 128), lambda i: (0, i, 0))]
    out_shape = [jax.ShapeDtypeStruct((T, EP, 64), _bf16),
                 jax.ShapeDtypeStruct((T, EP, 128), _f32)]
    return pl.pallas_call(
        _enc1_body, grid=grid, in_specs=in_specs, out_specs=out_specs,
        out_shape=out_shape)(xg, xg, eaP, *ws)


def _mega(ea2, g2flat, xg, gum, ws):
    bm = 1024
    grid = (EP // bm,)
    nb = EP // bm
    in_specs = [pl.BlockSpec((T, bm, 64), lambda i: (0, i, 0))]
    in_specs += [
        pl.BlockSpec((bm, 64), functools.partial(
            lambda tp, i: (tp * nb + i, 0), tp))
        for tp in range(2 * T)
    ]
    in_specs += [
        _edge_spec_d(bm), _edge_spec_s(bm),
        pl.BlockSpec((bm, 16), lambda i: (i, 0)),
    ]
    in_specs += [_full_spec(w.shape) for w in ws]
    out_specs = pl.BlockSpec((T - 1, bm, 128), lambda i: (0, i, 0))
    out_shape = jax.ShapeDtypeStruct((T - 1, EP, 128), _f32)
    return pl.pallas_call(
        _mega_body, grid=grid, in_specs=in_specs, out_specs=out_specs,
        out_shape=out_shape)(ea2, *([g2flat] * (2 * T)), xg, xg, gum, *ws)


def _addp(p):
    tn = p.shape[1]
    grid = (tn, NACC // BN)
    return pl.pallas_call(
        _addp_body, grid=grid,
        in_specs=[pl.BlockSpec((NC, 1, BN, 128), lambda t, i: (0, t, i, 0))],
        out_specs=pl.BlockSpec((BN, 64),
                               lambda t, i: (t * (NACC // BN) + i, 0)),
        out_shape=jax.ShapeDtypeStruct((tn * NACC, 64), _bf16))(p)


def _node(xnP, p2, ws):
    grid = (NACC // BN,)
    in_specs = [pl.BlockSpec((BN, 20), lambda i: (i, 0)),
                pl.BlockSpec((NC, T - 1, BN, 128), lambda i: (0, 0, i, 0))]
    in_specs += [_full_spec(w.shape) for w in ws]
    out_specs = pl.BlockSpec((BN, T - 1, DS), lambda i: (i, 0, 0))
    out_shape = jax.ShapeDtypeStruct((NACC, T - 1, DS), _f32)
    return pl.pallas_call(
        _node_body, grid=grid, in_specs=in_specs, out_specs=out_specs,
        out_shape=out_shape)(xnP, p2, *ws)


def _b2(b):
    return b.reshape(1, -1)


def _enc1_weights(params):
    wnp, bnp = params['node_proj']
    wep, bep = params['edge_proj']
    p1 = params['mpnn1']
    we1, be1 = p1['e1']
    we2, be2 = p1['e2']
    wn1, bn1 = p1['n1']
    wn2, bn2 = p1['n2']
    we1t = we1.T
    wn1t = wn1.T
    return [wnp.T, _b2(bnp), wep.T, _b2(bep),
            we1t[:H], we1t[H:2 * H], we1t[2 * H:], _b2(be1),
            we2.T, _b2(be2),
            wn1t[:H], wn1t[H:], _b2(bn1),
            jnp.pad(wn2.T, ((0, 0), (0, 64))), _b2(jnp.pad(bn2, (0, 64)))]


def _gru_weights(p):
    wih, whh, bih, bhh = p
    wit = wih.T
    wht = whh.T
    brz = _b2(bih[:2 * H] + bhh[:2 * H])
    return [wit[:, :2 * H], wit[:, 2 * H:], wht[:, :2 * H], wht[:, 2 * H:],
            brz, _b2(bih[2 * H:]), _b2(bhh[2 * H:])]


def _mega_weights(params):
    p2 = params['mpnn2']
    we1, be1 = p2['e1']
    we2, be2 = p2['e2']
    wh, bh = params['edge_head']
    we1t = we1.T
    ws = [we1t[:H], we1t[H:2 * H], we1t[2 * H:], _b2(be1),
          we2.T, _b2(be2), wh.T, _b2(bh)]
    ws += _gru_weights(params['gru_f'])
    ws += _gru_weights(params['gru_b'])
    wo, bo = params['gru_out']
    wot = wo.T
    ws += [wot[:H], wot[H:], _b2(bo)]
    w1ds, w1ss, b1s, w2s, b2s = [], [], [], [], []
    for k in range(K):
        (w1, b1), (w2, b2) = params['dec']['msg'][k]
        w1t = w1.T
        w1ds.append(w1t[:DS])
        w1ss.append(w1t[DS:])
        b1s.append(b1)
        w2s.append(w2.T)
        b2s.append(b2)
    w2bd = jax.scipy.linalg.block_diag(*w2s)
    rep = jnp.kron(jnp.eye(K, dtype=_f32), jnp.ones((1, DH), _f32))
    summ = jnp.pad(jnp.concatenate([jnp.eye(DH, dtype=_f32)] * K, axis=0),
                   ((0, 0), (0, 64)))
    ws += [jnp.concatenate(w1ds, axis=1), jnp.concatenate(w1ss, axis=1),
           _b2(jnp.concatenate(b1s)), w2bd, _b2(jnp.concatenate(b2s)),
           rep, summ]
    return ws


def _node_weights(params):
    d = params['dec']
    wo1, bo1 = d['o1']
    wo2, bo2 = d['o2']
    wo3, bo3 = d['o3']
    wo1t = wo1.T
    return [wo1t[:DS], wo1t[DS:], _b2(bo1), wo2.T, _b2(bo2), wo3.T, _b2(bo3)]


def kernel(x, edge_attr, edge_index, params):
    src = edge_index[0].astype(jnp.int32)
    dst = edge_index[1].astype(jnp.int32)
    epad = EP - E

    x32 = jnp.pad(x, ((0, 0), (0, 32 - T * DS))).astype(_bf16)
    eaP = jnp.pad(edge_attr, ((0, epad), (0, 0)))
    dst0 = jnp.pad(dst, (0, epad))
    src0 = jnp.pad(src, (0, epad))
    dstD = jnp.pad(dst, (0, epad), constant_values=N)

    idx1 = jnp.concatenate([dst0, src0]).reshape(NW, 80, 128)
    idxS = dstD.reshape(NW, 40, 128)
    idx2 = jnp.concatenate(
        [arr for t in range(T) for arr in (dst0 + t * NACC, src0 + t * NACC)]
    ).reshape(NW, 400, 128)
    zeros_acc = jnp.zeros((NACC, 128), _f32)

    u = jax.random.uniform(jax.random.key(42), (E, T, K),
                           minval=1e-10, maxval=1.0)
    g = -jnp.log(-jnp.log(u))
    gum = jnp.pad(g.reshape(E, T * K)[:, :4 * K], ((0, epad), (0, 0)))

    xg = _sc_gather(x32, idx1, 32, 80, 8, _bf16)

    ea2, m = _enc1(xg, eaP, _enc1_weights(params))

    pP = _sc_scatter(m, idxS, zeros_acc)
    x2tab = _addp(pP)

    g2 = _sc_gather(x2tab, idx2, 64, 400, 4, _bf16)

    msgs = _mega(ea2, g2, xg, gum, _mega_weights(params))

    p2P = _sc_scatter(msgs, idxS, zeros_acc)
    xnP = jnp.pad(x, ((0, NACC - N), (0, 0)))
    preds = _node(xnP, p2P, _node_weights(params))
    return preds[:N]


# speedup vs baseline: 1.0313x; 1.0313x over previous
import functools

import jax
import jax.numpy as jnp
from jax import lax
from jax.experimental import pallas as pl
from jax.experimental.pallas import tpu as pltpu
from jax.experimental.pallas import tpu_sc as plsc

N = 10000
E = 160000
T = 5
DS = 4
H = 64
K = 4
DH = 64
TEMP = 0.5

NC = 2
NS = 16
NW = NC * NS
EP = 163840
NACC = 10240
BE = 2048
BN = 2048

_f32 = jnp.float32
_bf16 = jnp.bfloat16


def _relu(v):
    return jnp.maximum(v, 0.0)


def _dot(a, b):
    return jnp.dot(a.astype(_bf16), b.astype(_bf16),
                   preferred_element_type=_f32)


def _dotf(a, b):
    return jnp.dot(a, b, preferred_element_type=_f32)


def _full_spec(shape):
    return pl.BlockSpec(shape, lambda i: (0,) * len(shape))


def _sc_gather(tab, idx, d, nch, gch, dtype=_f32):
    rows_w = nch * 128
    grows = gch * 128
    ngroups = nch // gch
    mesh = plsc.VectorSubcoreMesh(core_axis_name="c", subcore_axis_name="s")

    @functools.partial(
        pl.kernel,
        out_type=jax.ShapeDtypeStruct((NW * rows_w, d), dtype),
        mesh=mesh,
        compiler_params=pltpu.CompilerParams(use_tc_tiling_on_sc=False),
        scratch_types=[
            pltpu.VMEM((nch, 128), jnp.int32),
            pltpu.VMEM((grows, d), dtype),
            pltpu.SemaphoreType.DMA,
        ],
    )
    def body(tab_h, idx_h, out_h, idxv, rowsv, sem):
        wid = lax.axis_index("s") * NC + lax.axis_index("c")
        pltpu.sync_copy(idx_h.at[wid], idxv)
        base = wid * rows_w

        def per_group(g, carry):
            descs = []
            for c in range(gch):
                descs.append(pltpu.async_copy(
                    tab_h.at[idxv.at[g * gch + c]],
                    rowsv.at[pl.ds(c * 128, 128)], sem))
            for dsc in descs:
                dsc.wait()
            pltpu.sync_copy(rowsv, out_h.at[pl.ds(base + g * grows, grows)])
            return carry

        lax.fori_loop(0, ngroups, per_group, 0)

    return body(tab, idx)


def _sc_scatter(msgs, idx, zeros_acc):
    tn = msgs.shape[0]
    rows_w = EP // NW
    acc_sl = NACC // NS
    mesh = plsc.VectorSubcoreMesh(core_axis_name="c", subcore_axis_name="s")

    @functools.partial(
        pl.kernel,
        out_type=jax.ShapeDtypeStruct((NC, tn, NACC, 128), _f32),
        mesh=mesh,
        compiler_params=pltpu.CompilerParams(use_tc_tiling_on_sc=False),
        scratch_types=[
            pltpu.VMEM((40, 128), jnp.int32),
            pltpu.VMEM((256, 128), _f32),
            pltpu.VMEM_SHARED((NACC, 128), _f32),
        ],
    )
    def body(m_h, idx_h, z_h, out_h, idxv, datav, acc):
        cid = lax.axis_index("c")
        sid = lax.axis_index("s")
        wid = sid * NC + cid
        pltpu.sync_copy(idx_h.at[wid], idxv)
        base = wid * rows_w

        def per_t(t, carry):
            pltpu.sync_copy(z_h.at[pl.ds(sid * acc_sl, acc_sl)],
                            acc.at[pl.ds(sid * acc_sl, acc_sl)])
            plsc.subcore_barrier()

            def per_group(g, c2):
                pltpu.sync_copy(m_h.at[t].at[pl.ds(base + g * 256, 256)],
                                datav)
                for c in range(2):
                    pltpu.sync_copy(datav.at[pl.ds(c * 128, 128)],
                                    acc.at[idxv.at[g * 2 + c]], add=True)
                return c2

            lax.fori_loop(0, 20, per_group, 0)
            plsc.subcore_barrier()
            pltpu.sync_copy(acc.at[pl.ds(sid * acc_sl, acc_sl)],
                            out_h.at[cid].at[t].at[pl.ds(sid * acc_sl, acc_sl)])
            plsc.subcore_barrier()
            return carry

        lax.fori_loop(0, tn, per_t, 0)

    return body(msgs, idx, zeros_acc)


def _enc1_body(xgd, xgs, ea,
               wnp, bnp, wep, bep,
               a1, b1, c1, be1, we2, be2,
               d1, d2, bn1, wn2, bn2,
               ea2_o, m_o):
    xd = xgd[...]
    xs_ = xgs[...]
    eaa = ea[...]
    for t in range(T):
        sl = slice(4 * t, 4 * t + 4)
        xi = _relu(_dot(xd[:, sl], wnp[...]) + bnp[...])
        xj = _relu(_dot(xs_[:, sl], wnp[...]) + bnp[...])
        he = _relu(_dot(eaa[:, sl], wep[...]) + bep[...])
        h = _relu(_dot(xi, a1[...]) + _dot(xj, b1[...]) + _dot(he, c1[...])
                  + be1[...])
        ea2 = _dot(h, we2[...]) + be2[...]
        mm = _dot(_relu(_dot(xi, d1[...]) + _dot(ea2, d2[...]) + bn1[...]),
                  wn2[...]) + bn2[...]
        ea2_o[t] = ea2.astype(_bf16)
        m_o[t] = mm


def _mega_body(ea2, g0d, g0s, g1d, g1s, g2d, g2s, g3d, g3s, g4d, g4s,
               xgd, xgs, gum,
               a2, b2, c2, be1, we2, be2, wh, bh,
               wir_f, wiz_f, win_f, whr_f, whz_f, whn_f,
               bir_f, biz_f, bin_f, bhr_f, bhz_f, bhn_f,
               wir_b, wiz_b, win_b, whr_b, whz_b, whn_b,
               bir_b, biz_b, bin_b, bhr_b, bhz_b, bhn_b,
               wof, wob, bo,
               w1d, w1s, b1c, w2bd, b2c, rep, summ,
               msg_o):
    gds = [g0d, g1d, g2d, g3d, g4d]
    gss = [g0s, g1s, g2s, g3s, g4s]
    logits = []
    for t in range(T):
        xi2 = gds[t][...]
        xj2 = gss[t][...]
        e2t = ea2[t]
        h = _relu(_dot(xi2, a2[...]) + _dot(xj2, b2[...]) + _dot(e2t, c2[...])
                  + be1[...])
        ea3 = _dot(h, we2[...]) + be2[...]
        logits.append(_dot(ea3, wh[...]) + bh[...])

    def gru_step(xt, hprev, wir, wiz, win, whr, whz, whn,
                 bir, biz, binn, bhr, bhz, bhn):
        r = jax.nn.sigmoid(_dot(xt, wir[...]) + bir[...]
                           + _dot(hprev, whr[...]) + bhr[...])
        z = jax.nn.sigmoid(_dot(xt, wiz[...]) + biz[...]
                           + _dot(hprev, whz[...]) + bhz[...])
        c = jnp.tanh(_dot(xt, win[...]) + binn[...]
                     + r * (_dot(hprev, whn[...]) + bhn[...]))
        return (1.0 - z) * c + z * hprev

    nb = logits[0].shape[0]
    hf = jnp.zeros((nb, H), _f32)
    yf = []
    for t in range(T):
        hf = gru_step(logits[t], hf, wir_f, wiz_f, win_f, whr_f, whz_f,
                      whn_f, bir_f, biz_f, bin_f, bhr_f, bhz_f, bhn_f)
        yf.append(hf)
    hb = jnp.zeros((nb, H), _f32)
    yb = [None] * T
    for t in reversed(range(T)):
        hb = gru_step(logits[t], hb, wir_b, wiz_b, win_b, whr_b, whz_b,
                      whn_b, bir_b, biz_b, bin_b, bhr_b, bhz_b, bhn_b)
        yb[t] = hb

    xd = xgd[...]
    xs_ = xgs[...]
    gu = gum[...]
    for t in range(T - 1):
        sm = _dot(yf[t], wof[...]) + _dot(yb[t], wob[...]) + bo[...]
        a = (sm + gu[:, 4 * t:4 * t + 4]) * (1.0 / TEMP)
        ex = jnp.exp(a)
        pr = ex / jnp.sum(ex, axis=1, keepdims=True)
        pbig = _dotf(pr, rep[...])
        sl = slice(4 * t, 4 * t + 4)
        xdt = xd[:, sl]
        xst = xs_[:, sl]
        h1 = _relu(_dot(xdt, w1d[...]) + _dot(xst, w1s[...]) + b1c[...])
        hc = _relu(_dot(h1, w2bd[...]) + b2c[...])
        msg_o[t] = _dotf(hc * pbig, summ[...])


def _addp_body(p, o):
    o[...] = (p[0, 0, :, :64] + p[1, 0, :, :64]).astype(_bf16)


def _node_body(xr, p2, o1a, o1b, bo1, w2, bo2, w3, bo3, out):
    xx = xr[...]
    for t in range(T - 1):
        aggr = p2[0, t, :, :64] + p2[1, t, :, :64]
        xt = xx[:, 4 * t:4 * t + 4]
        h = _relu(_dotf(xt, o1a[...]) + _dotf(aggr, o1b[...]) + bo1[...])
        h = _relu(_dotf(h, w2[...]) + bo2[...])
        delta = _dotf(h, w3[...]) + bo3[...]
        out[:, t, :] = xt + delta


def _edge_spec_d(be):
    return pl.BlockSpec((be, 32), lambda i: (i, 0))


def _edge_spec_s(be):
    return pl.BlockSpec((be, 32), lambda i: (EP // be + i, 0))


def _enc1(xg, eaP, ws):
    grid = (EP // BE,)
    in_specs = [_edge_spec_d(BE), _edge_spec_s(BE),
                pl.BlockSpec((BE, 20), lambda i: (i, 0))]
    in_specs += [_full_spec(w.shape) for w in ws]
    out_specs = [pl.BlockSpec((T, BE, 64), lambda i: (0, i, 0)),
                 pl.BlockSpec((T, BE, 128), lambda i: (0, i, 0))]
    out_shape = [jax.ShapeDtypeStruct((T, EP, 64), _bf16),
                 jax.ShapeDtypeStruct((T, EP, 128), _f32)]
    return pl.pallas_call(
        _enc1_body, grid=grid, in_specs=in_specs, out_specs=out_specs,
        out_shape=out_shape)(xg, xg, eaP, *ws)


def _mega(ea2, g2flat, xg, gum, ws):
    bm = 1024
    grid = (EP // bm,)
    nb = EP // bm
    in_specs = [pl.BlockSpec((T, bm, 64), lambda i: (0, i, 0))]
    in_specs += [
        pl.BlockSpec((bm, 64), functools.partial(
            lambda tp, i: (tp * nb + i, 0), tp))
        for tp in range(2 * T)
    ]
    in_specs += [
        _edge_spec_d(bm), _edge_spec_s(bm),
        pl.BlockSpec((bm, 16), lambda i: (i, 0)),
    ]
    in_specs += [_full_spec(w.shape) for w in ws]
    out_specs = pl.BlockSpec((T - 1, bm, 128), lambda i: (0, i, 0))
    out_shape = jax.ShapeDtypeStruct((T - 1, EP, 128), _f32)
    return pl.pallas_call(
        _mega_body, grid=grid, in_specs=in_specs, out_specs=out_specs,
        out_shape=out_shape)(ea2, *([g2flat] * (2 * T)), xg, xg, gum, *ws)


def _addp(p):
    tn = p.shape[1]
    grid = (tn, NACC // BN)
    return pl.pallas_call(
        _addp_body, grid=grid,
        in_specs=[pl.BlockSpec((NC, 1, BN, 128), lambda t, i: (0, t, i, 0))],
        out_specs=pl.BlockSpec((BN, 64),
                               lambda t, i: (t * (NACC // BN) + i, 0)),
        out_shape=jax.ShapeDtypeStruct((tn * NACC, 64), _bf16))(p)


def _node(xnP, p2, ws):
    grid = (NACC // BN,)
    in_specs = [pl.BlockSpec((BN, 20), lambda i: (i, 0)),
                pl.BlockSpec((NC, T - 1, BN, 128), lambda i: (0, 0, i, 0))]
    in_specs += [_full_spec(w.shape) for w in ws]
    out_specs = pl.BlockSpec((BN, T - 1, DS), lambda i: (i, 0, 0))
    out_shape = jax.ShapeDtypeStruct((NACC, T - 1, DS), _f32)
    return pl.pallas_call(
        _node_body, grid=grid, in_specs=in_specs, out_specs=out_specs,
        out_shape=out_shape)(xnP, p2, *ws)


def _b2(b):
    return b.reshape(1, -1)


def _enc1_weights(params):
    wnp, bnp = params['node_proj']
    wep, bep = params['edge_proj']
    p1 = params['mpnn1']
    we1, be1 = p1['e1']
    we2, be2 = p1['e2']
    wn1, bn1 = p1['n1']
    wn2, bn2 = p1['n2']
    we1t = we1.T
    wn1t = wn1.T
    return [wnp.T, _b2(bnp), wep.T, _b2(bep),
            we1t[:H], we1t[H:2 * H], we1t[2 * H:], _b2(be1),
            we2.T, _b2(be2),
            wn1t[:H], wn1t[H:], _b2(bn1),
            jnp.pad(wn2.T, ((0, 0), (0, 64))), _b2(jnp.pad(bn2, (0, 64)))]


def _gru_weights(p):
    wih, whh, bih, bhh = p
    out = []
    for wmat in (wih, whh):
        wt = wmat.T
        out += [wt[:, :H], wt[:, H:2 * H], wt[:, 2 * H:]]
    for bvec in (bih, bhh):
        out += [_b2(bvec[:H]), _b2(bvec[H:2 * H]), _b2(bvec[2 * H:])]
    return out


def _mega_weights(params):
    p2 = params['mpnn2']
    we1, be1 = p2['e1']
    we2, be2 = p2['e2']
    wh, bh = params['edge_head']
    we1t = we1.T
    ws = [we1t[:H], we1t[H:2 * H], we1t[2 * H:], _b2(be1),
          we2.T, _b2(be2), wh.T, _b2(bh)]
    ws += _gru_weights(params['gru_f'])
    ws += _gru_weights(params['gru_b'])
    wo, bo = params['gru_out']
    wot = wo.T
    ws += [wot[:H], wot[H:], _b2(bo)]
    w1ds, w1ss, b1s, w2s, b2s = [], [], [], [], []
    for k in range(K):
        (w1, b1), (w2, b2) = params['dec']['msg'][k]
        w1t = w1.T
        w1ds.append(w1t[:DS])
        w1ss.append(w1t[DS:])
        b1s.append(b1)
        w2s.append(w2.T)
        b2s.append(b2)
    w2bd = jax.scipy.linalg.block_diag(*w2s)
    rep = jnp.kron(jnp.eye(K, dtype=_f32), jnp.ones((1, DH), _f32))
    summ = jnp.pad(jnp.concatenate([jnp.eye(DH, dtype=_f32)] * K, axis=0),
                   ((0, 0), (0, 64)))
    ws += [jnp.concatenate(w1ds, axis=1), jnp.concatenate(w1ss, axis=1),
           _b2(jnp.concatenate(b1s)), w2bd, _b2(jnp.concatenate(b2s)),
           rep, summ]
    return ws


def _node_weights(params):
    d = params['dec']
    wo1, bo1 = d['o1']
    wo2, bo2 = d['o2']
    wo3, bo3 = d['o3']
    wo1t = wo1.T
    return [wo1t[:DS], wo1t[DS:], _b2(bo1), wo2.T, _b2(bo2), wo3.T, _b2(bo3)]


def kernel(x, edge_attr, edge_index, params):
    src = edge_index[0].astype(jnp.int32)
    dst = edge_index[1].astype(jnp.int32)
    epad = EP - E

    x32 = jnp.pad(x, ((0, 0), (0, 32 - T * DS))).astype(_bf16)
    eaP = jnp.pad(edge_attr, ((0, epad), (0, 0)))
    dst0 = jnp.pad(dst, (0, epad))
    src0 = jnp.pad(src, (0, epad))
    dstD = jnp.pad(dst, (0, epad), constant_values=N)

    idx1 = jnp.concatenate([dst0, src0]).reshape(NW, 80, 128)
    idxS = dstD.reshape(NW, 40, 128)
    idx2 = jnp.concatenate(
        [arr for t in range(T) for arr in (dst0 + t * NACC, src0 + t * NACC)]
    ).reshape(NW, 400, 128)
    zeros_acc = jnp.zeros((NACC, 128), _f32)

    u = jax.random.uniform(jax.random.key(42), (E, T, K),
                           minval=1e-10, maxval=1.0)
    g = -jnp.log(-jnp.log(u))
    gum = jnp.pad(g.reshape(E, T * K)[:, :4 * K], ((0, epad), (0, 0)))

    xg = _sc_gather(x32, idx1, 32, 80, 8, _bf16)

    ea2, m = _enc1(xg, eaP, _enc1_weights(params))

    pP = _sc_scatter(m, idxS, zeros_acc)
    x2tab = _addp(pP)

    g2 = _sc_gather(x2tab, idx2, 64, 400, 4, _bf16)

    msgs = _mega(ea2, g2, xg, gum, _mega_weights(params))

    p2P = _sc_scatter(msgs, idxS, zeros_acc)
    xnP = jnp.pad(x, ((0, NACC - N), (0, 0)))
    preds = _node(xnP, p2P, _node_weights(params))
    return preds[:N]
